# baseline (speedup 1.0000x reference)
import functools

import jax
import jax.numpy as jnp
from jax import lax
from jax.experimental import pallas as pl
from jax.experimental.pallas import tpu as pltpu
from jax.experimental.pallas import tpu_sc as plsc

MODEL_DIM = 1024
BATCH = 4
SEQ = 8192
TOTAL = BATCH * SEQ

NUM_CORES = 2
NUM_SUBCORES = 16
NUM_WORKERS = NUM_CORES * NUM_SUBCORES

PER_WORKER = TOTAL // NUM_WORKERS
CHUNK = 32
NBUF = 3
NUM_CHUNKS = PER_WORKER // CHUNK
NUM_GROUPS = NUM_CHUNKS // NBUF
VPR = MODEL_DIM // 16

_SPLAT_DNUMS = lax.GatherDimensionNumbers(
    offset_dims=(), collapsed_slice_dims=(0,), start_index_map=(0,)
)


def _vsplat(vec, lane):
    idx = jnp.full((16, 1), lane, jnp.int32)
    return lax.gather(
        vec, idx, _SPLAT_DNUMS, (1,),
        mode=lax.GatherScatterMode.PROMISE_IN_BOUNDS,
    )


@functools.partial(
    pl.kernel,
    mesh=plsc.VectorSubcoreMesh(core_axis_name="c", subcore_axis_name="s"),
    out_type=jax.ShapeDtypeStruct((TOTAL, MODEL_DIM), jnp.float32),
    scratch_types=[
        pltpu.VMEM((PER_WORKER,), jnp.int32),
        pltpu.VMEM((2 * MODEL_DIM,), jnp.float32),
        pltpu.VMEM((MODEL_DIM,), jnp.int32),
        pltpu.VMEM((CHUNK, MODEL_DIM), jnp.float32),
        pltpu.VMEM((CHUNK, MODEL_DIM), jnp.float32),
        pltpu.VMEM((CHUNK, MODEL_DIM), jnp.float32),
        pltpu.SemaphoreType.DMA,
        pltpu.SemaphoreType.DMA,
        pltpu.SemaphoreType.DMA,
    ],
)
def _emb_lookup(idx_hbm, table_hbm, out_hbm, idx_v, tab_v, x_v, buf0, buf1, buf2, s0, s1, s2):
    wid = lax.axis_index("s") * NUM_CORES + lax.axis_index("c")
    base = wid * PER_WORKER
    wpb = SEQ // PER_WORKER
    pltpu.sync_copy(
        idx_hbm.at[wid // wpb, pl.ds((wid % wpb) * PER_WORKER, PER_WORKER)], idx_v
    )
    pltpu.sync_copy(table_hbm.at[0], tab_v.at[pl.ds(0, MODEL_DIM)])
    pltpu.sync_copy(table_hbm.at[1], tab_v.at[pl.ds(MODEL_DIM, MODEL_DIM)])

    bufs = (buf0, buf1, buf2)
    sems = (s0, s1, s2)
    zeros16 = jnp.zeros((16,), jnp.int32)

    def x_body(j, carry):
        o = j * 16
        r0 = tab_v[pl.ds(o, 16)].view(jnp.int32)
        r1 = tab_v[pl.ds(MODEL_DIM + o, 16)].view(jnp.int32)
        x_v[pl.ds(o, 16)] = r0 ^ r1
        return carry

    lax.fori_loop(0, VPR, x_body, 0)

    NPASS = 4
    PART = MODEL_DIM // NPASS
    KP = PART // 16

    def build_chunk(c, buf):
        for h in range(NPASS):
            x = [x_v[pl.ds(h * PART + k * 16, 16)] for k in range(KP)]
            r0s = [
                tab_v[pl.ds(h * PART + k * 16, 16)].view(jnp.int32)
                for k in range(KP)
            ]

            @plsc.parallel_loop(0, CHUNK, unroll=4)
            def row_body(r):
                g16 = (r // 16) * 16
                lane = r - g16
                idxv = idx_v[pl.ds(c * CHUNK + g16, 16)]
                m = zeros16 - _vsplat(idxv, lane)
                for k in range(KP):
                    sel = (m & x[k]) ^ r0s[k]
                    buf[r, pl.ds(h * PART + k * 16, 16)] = sel.view(jnp.float32)

    def out_slice(c):
        return out_hbm.at[pl.ds(base + c * CHUNK, CHUNK)]

    for b in range(NBUF):
        build_chunk(b, bufs[b])
        pltpu.make_async_copy(bufs[b], out_slice(b), sems[b]).start()

    def group_loop(g, carry):
        for b in range(NBUF):
            c = g * NBUF + b
            pltpu.make_async_copy(bufs[b], out_slice(c), sems[b]).wait()
            build_chunk(c, bufs[b])
            pltpu.make_async_copy(bufs[b], out_slice(c), sems[b]).start()
        return carry

    lax.fori_loop(1, NUM_GROUPS, group_loop, 0)

    for b in range(NBUF):
        pltpu.make_async_copy(bufs[b], out_slice(0), sems[b]).wait()


def kernel(speaker_id, table):
    out = _emb_lookup(speaker_id.astype(jnp.int32), table)
    return out.reshape(BATCH, SEQ, MODEL_DIM)


# speedup vs baseline: 1.1876x; 1.1876x over previous
import functools

import jax
import jax.numpy as jnp
from jax import lax
from jax.experimental import pallas as pl
from jax.experimental.pallas import tpu as pltpu
from jax.experimental.pallas import tpu_sc as plsc

MODEL_DIM = 1024
BATCH = 4
SEQ = 8192
TOTAL = BATCH * SEQ

NUM_CORES = 2
NUM_SUBCORES = 16
NUM_WORKERS = NUM_CORES * NUM_SUBCORES

PER_WORKER = TOTAL // NUM_WORKERS
CHUNK = 16
NBUF = 4
NUM_CHUNKS = PER_WORKER // CHUNK
NUM_GROUPS = NUM_CHUNKS // NBUF
VPR = MODEL_DIM // 16

_SPLAT_DNUMS = lax.GatherDimensionNumbers(
    offset_dims=(), collapsed_slice_dims=(0,), start_index_map=(0,)
)


def _vsplat(vec, lane):
    idx = jnp.full((16, 1), lane, jnp.int32)
    return lax.gather(
        vec, idx, _SPLAT_DNUMS, (1,),
        mode=lax.GatherScatterMode.PROMISE_IN_BOUNDS,
    )


@functools.partial(
    pl.kernel,
    mesh=plsc.VectorSubcoreMesh(core_axis_name="c", subcore_axis_name="s"),
    out_type=jax.ShapeDtypeStruct((TOTAL, MODEL_DIM), jnp.float32),
    scratch_types=[
        pltpu.VMEM((PER_WORKER,), jnp.int32),
        pltpu.VMEM((2 * MODEL_DIM,), jnp.float32),
        pltpu.VMEM((MODEL_DIM,), jnp.int32),
        pltpu.VMEM((CHUNK, MODEL_DIM), jnp.float32),
        pltpu.VMEM((CHUNK, MODEL_DIM), jnp.float32),
        pltpu.VMEM((CHUNK, MODEL_DIM), jnp.float32),
        pltpu.VMEM((CHUNK, MODEL_DIM), jnp.float32),
        pltpu.SemaphoreType.DMA,
        pltpu.SemaphoreType.DMA,
        pltpu.SemaphoreType.DMA,
        pltpu.SemaphoreType.DMA,
    ],
)
def _emb_lookup(idx_hbm, table_hbm, out_hbm, idx_v, tab_v, x_v, buf0, buf1, buf2, buf3, s0, s1, s2, s3):
    wid = lax.axis_index("s") * NUM_CORES + lax.axis_index("c")
    base = wid * PER_WORKER
    wpb = SEQ // PER_WORKER
    pltpu.sync_copy(
        idx_hbm.at[wid // wpb, pl.ds((wid % wpb) * PER_WORKER, PER_WORKER)], idx_v
    )
    pltpu.sync_copy(table_hbm.at[0], tab_v.at[pl.ds(0, MODEL_DIM)])
    pltpu.sync_copy(table_hbm.at[1], tab_v.at[pl.ds(MODEL_DIM, MODEL_DIM)])

    bufs = (buf0, buf1, buf2, buf3)
    sems = (s0, s1, s2, s3)
    zeros16 = jnp.zeros((16,), jnp.int32)

    def x_body(j, carry):
        o = j * 16
        r0 = tab_v[pl.ds(o, 16)].view(jnp.int32)
        r1 = tab_v[pl.ds(MODEL_DIM + o, 16)].view(jnp.int32)
        x_v[pl.ds(o, 16)] = r0 ^ r1
        return carry

    lax.fori_loop(0, VPR, x_body, 0)

    NPASS = 4
    PART = MODEL_DIM // NPASS
    KP = PART // 16

    def build_chunk(c, buf):
        for h in range(NPASS):
            x = [x_v[pl.ds(h * PART + k * 16, 16)] for k in range(KP)]
            r0s = [
                tab_v[pl.ds(h * PART + k * 16, 16)].view(jnp.int32)
                for k in range(KP)
            ]

            @plsc.parallel_loop(0, CHUNK, unroll=2)
            def row_body(r):
                g16 = (r // 16) * 16
                lane = r - g16
                idxv = idx_v[pl.ds(c * CHUNK + g16, 16)]
                m = zeros16 - _vsplat(idxv, lane)
                for k in range(KP):
                    sel = (m & x[k]) ^ r0s[k]
                    buf[r, pl.ds(h * PART + k * 16, 16)] = sel.view(jnp.float32)

    def out_slice(c):
        return out_hbm.at[pl.ds(base + c * CHUNK, CHUNK)]

    for b in range(NBUF):
        build_chunk(b, bufs[b])
        pltpu.make_async_copy(bufs[b], out_slice(b), sems[b]).start()

    def group_loop(g, carry):
        for b in range(NBUF):
            c = g * NBUF + b
            pltpu.make_async_copy(bufs[b], out_slice(c), sems[b]).wait()
            build_chunk(c, bufs[b])
            pltpu.make_async_copy(bufs[b], out_slice(c), sems[b]).start()
        return carry

    lax.fori_loop(1, NUM_GROUPS, group_loop, 0)

    for b in range(NBUF):
        pltpu.make_async_copy(bufs[b], out_slice(0), sems[b]).wait()


def kernel(speaker_id, table):
    out = _emb_lookup(speaker_id.astype(jnp.int32), table)
    return out.reshape(BATCH, SEQ, MODEL_DIM)


# speedup vs baseline: 1.6044x; 1.3510x over previous
import functools

import jax
import jax.numpy as jnp
from jax import lax
from jax.experimental import pallas as pl
from jax.experimental.pallas import tpu as pltpu
from jax.experimental.pallas import tpu_sc as plsc

MODEL_DIM = 1024
BATCH = 4
SEQ = 8192
TOTAL = BATCH * SEQ

NUM_CORES = 2
NUM_SUBCORES = 16
NUM_WORKERS = NUM_CORES * NUM_SUBCORES

PER_WORKER = TOTAL // NUM_WORKERS
CHUNK = 32
NBUF = 3
NUM_CHUNKS = PER_WORKER // CHUNK
NUM_GROUPS = NUM_CHUNKS // NBUF
VPR = MODEL_DIM // 16

_SPLAT_DNUMS = lax.GatherDimensionNumbers(
    offset_dims=(), collapsed_slice_dims=(0,), start_index_map=(0,)
)


def _vsplat(vec, lane):
    idx = jnp.full((16, 1), lane, jnp.int32)
    return lax.gather(
        vec, idx, _SPLAT_DNUMS, (1,),
        mode=lax.GatherScatterMode.PROMISE_IN_BOUNDS,
    )


@functools.partial(
    pl.kernel,
    mesh=plsc.VectorSubcoreMesh(core_axis_name="c", subcore_axis_name="s"),
    out_type=jax.ShapeDtypeStruct((TOTAL, MODEL_DIM), jnp.float32),
    scratch_types=[
        pltpu.VMEM((PER_WORKER,), jnp.int32),
        pltpu.VMEM((2 * MODEL_DIM,), jnp.float32),
        pltpu.VMEM((MODEL_DIM,), jnp.int32),
        pltpu.VMEM((CHUNK, MODEL_DIM), jnp.float32),
        pltpu.VMEM((CHUNK, MODEL_DIM), jnp.float32),
        pltpu.VMEM((CHUNK, MODEL_DIM), jnp.float32),
        pltpu.SemaphoreType.DMA,
        pltpu.SemaphoreType.DMA,
        pltpu.SemaphoreType.DMA,
    ],
)
def _emb_lookup(idx_hbm, table_hbm, out_hbm, idx_v, tab_v, x_v, buf0, buf1, buf2, s0, s1, s2):
    wid = lax.axis_index("s") * NUM_CORES + lax.axis_index("c")
    base = wid * PER_WORKER
    wpb = SEQ // PER_WORKER
    pltpu.sync_copy(
        idx_hbm.at[wid // wpb, pl.ds((wid % wpb) * PER_WORKER, PER_WORKER)], idx_v
    )
    pltpu.sync_copy(table_hbm.at[0], tab_v.at[pl.ds(0, MODEL_DIM)])
    pltpu.sync_copy(table_hbm.at[1], tab_v.at[pl.ds(MODEL_DIM, MODEL_DIM)])

    bufs = (buf0, buf1, buf2)
    sems = (s0, s1, s2)
    zeros16 = jnp.zeros((16,), jnp.int32)

    def x_body(j, carry):
        o = j * 16
        r0 = tab_v[pl.ds(o, 16)].view(jnp.int32)
        r1 = tab_v[pl.ds(MODEL_DIM + o, 16)].view(jnp.int32)
        x_v[pl.ds(o, 16)] = r0 ^ r1
        return carry

    lax.fori_loop(0, VPR, x_body, 0)

    NPASS = 4
    PART = MODEL_DIM // NPASS
    KP = PART // 16

    def build_chunk(c, buf):
        for h in range(NPASS):
            x = [x_v[pl.ds(h * PART + k * 16, 16)] for k in range(KP)]
            r0s = [
                tab_v[pl.ds(h * PART + k * 16, 16)].view(jnp.int32)
                for k in range(KP)
            ]

            @plsc.parallel_loop(0, CHUNK, unroll=2)
            def row_body(r):
                g16 = (r // 16) * 16
                lane = r - g16
                idxv = idx_v[pl.ds(c * CHUNK + g16, 16)]
                m = zeros16 - _vsplat(idxv, lane)
                for k in range(KP):
                    sel = (m & x[k]) ^ r0s[k]
                    buf[r, pl.ds(h * PART + k * 16, 16)] = sel.view(jnp.float32)

    def out_slice(c):
        return out_hbm.at[pl.ds(base + c * CHUNK, CHUNK)]

    for b in range(NBUF):
        build_chunk(b, bufs[b])
        pltpu.make_async_copy(bufs[b], out_slice(b), sems[b]).start()

    def group_loop(g, carry):
        for b in range(NBUF):
            c = g * NBUF + b
            pltpu.make_async_copy(bufs[b], out_slice(c), sems[b]).wait()
            build_chunk(c, bufs[b])
            pltpu.make_async_copy(bufs[b], out_slice(c), sems[b]).start()
        return carry

    lax.fori_loop(1, NUM_GROUPS, group_loop, 0)

    for b in range(NBUF):
        pltpu.make_async_copy(bufs[b], out_slice(0), sems[b]).wait()


def kernel(speaker_id, table):
    out = _emb_lookup(speaker_id.astype(jnp.int32), table)
    return out.reshape(BATCH, SEQ, MODEL_DIM)
